# baseline (device time: 11347 ns/iter reference)
import jax
import jax.numpy as jnp
from jax import lax
from jax.experimental import pallas as pl
from jax.experimental.pallas import tpu as pltpu

N_DEV = 4
P = 4
SEND_ORDER = (2, 1, 3)
WAIT_ORDER = (1, 3, 2)


def kernel(x):
    m, n = x.shape
    chunk = m // N_DEV
    sub = chunk // P
    npeer = N_DEV - 1

    def body(
        x_ref, out_ref,
        acc_buf, rs_recv,
        rs_send_sems, rs_recv_sems, ag_send_sems, ag_recv_sems,
    ):
        k = lax.axis_index("i")

        barrier_sem = pltpu.get_barrier_semaphore()
        for d in range(1, N_DEV):
            pl.semaphore_signal(
                barrier_sem, inc=1,
                device_id=((k + d) % N_DEV,),
                device_id_type=pl.DeviceIdType.MESH,
            )
        pl.semaphore_wait(barrier_sem, npeer)

        rs = [[None] * npeer for _ in range(P)]
        for p in range(P):
            for d in SEND_ORDER:
                t = (k + d) % N_DEV
                q_recv = p * npeer + (N_DEV - 1 - d)
                r = pltpu.make_async_remote_copy(
                    src_ref=x_ref.at[pl.ds(t * chunk + p * sub, sub), :],
                    dst_ref=rs_recv.at[q_recv],
                    send_sem=rs_send_sems.at[p * npeer + d - 1],
                    recv_sem=rs_recv_sems.at[q_recv],
                    device_id=(t,),
                    device_id_type=pl.DeviceIdType.MESH,
                )
                r.start()
                rs[p][d - 1] = r

        ag = [[None] * npeer for _ in range(P)]
        for p in range(P):
            reduced = x_ref[pl.ds(k * chunk + p * sub, sub), :]
            for d in WAIT_ORDER:
                rs[p][d - 1].wait_recv()
                reduced = reduced + rs_recv[p * npeer + (N_DEV - 1 - d), :, :]
            acc_buf[p, :, :] = reduced
            out_ref[pl.ds(k * chunk + p * sub, sub), :] = reduced
            for d in SEND_ORDER:
                t = (k + d) % N_DEV
                r = pltpu.make_async_remote_copy(
                    src_ref=acc_buf.at[p],
                    dst_ref=out_ref.at[pl.ds(k * chunk + p * sub, sub), :],
                    send_sem=ag_send_sems.at[p * npeer + d - 1],
                    recv_sem=ag_recv_sems.at[p * npeer + (N_DEV - 1 - d)],
                    device_id=(t,),
                    device_id_type=pl.DeviceIdType.MESH,
                )
                r.start()
                ag[p][d - 1] = r

        for p in range(P):
            for r in rs[p]:
                r.wait_send()
            for d in WAIT_ORDER:
                ag[p][d - 1].wait_recv()
            for r in ag[p]:
                r.wait_send()

    return pl.pallas_call(
        body,
        out_shape=jax.ShapeDtypeStruct((m, n), x.dtype),
        in_specs=[pl.BlockSpec(memory_space=pltpu.VMEM)],
        out_specs=pl.BlockSpec(memory_space=pltpu.VMEM),
        scratch_shapes=[
            pltpu.VMEM((P, sub, n), x.dtype),
            pltpu.VMEM((P * npeer, sub, n), x.dtype),
            pltpu.SemaphoreType.DMA((P * npeer,)),
            pltpu.SemaphoreType.DMA((P * npeer,)),
            pltpu.SemaphoreType.DMA((P * npeer,)),
            pltpu.SemaphoreType.DMA((P * npeer,)),
        ],
        compiler_params=pltpu.CompilerParams(collective_id=0),
    )(x)


# device time: 11131 ns/iter; 1.0194x vs baseline; 1.0194x over previous
import jax
import jax.numpy as jnp
from jax import lax
from jax.experimental import pallas as pl
from jax.experimental.pallas import tpu as pltpu

N_DEV = 4
P = 2
SEND_ORDER = (2, 1, 3)
WAIT_ORDER = (1, 3, 2)


def kernel(x):
    m, n = x.shape
    chunk = m // N_DEV
    sub = chunk // P
    npeer = N_DEV - 1

    def body(
        x_ref, out_ref,
        acc_buf, rs_recv,
        rs_send_sems, rs_recv_sems, ag_send_sems, ag_recv_sems,
    ):
        k = lax.axis_index("i")

        barrier_sem = pltpu.get_barrier_semaphore()
        for d in range(1, N_DEV):
            pl.semaphore_signal(
                barrier_sem, inc=1,
                device_id=((k + d) % N_DEV,),
                device_id_type=pl.DeviceIdType.MESH,
            )
        pl.semaphore_wait(barrier_sem, npeer)

        rs = [[None] * npeer for _ in range(P)]
        for p in range(P):
            for d in SEND_ORDER:
                t = (k + d) % N_DEV
                q_recv = p * npeer + (N_DEV - 1 - d)
                r = pltpu.make_async_remote_copy(
                    src_ref=x_ref.at[pl.ds(t * chunk + p * sub, sub), :],
                    dst_ref=rs_recv.at[q_recv],
                    send_sem=rs_send_sems.at[p * npeer + d - 1],
                    recv_sem=rs_recv_sems.at[q_recv],
                    device_id=(t,),
                    device_id_type=pl.DeviceIdType.MESH,
                )
                r.start()
                rs[p][d - 1] = r

        ag = [[None] * npeer for _ in range(P)]
        for p in range(P):
            reduced = x_ref[pl.ds(k * chunk + p * sub, sub), :]
            for d in WAIT_ORDER:
                rs[p][d - 1].wait_recv()
                reduced = reduced + rs_recv[p * npeer + (N_DEV - 1 - d), :, :]
            acc_buf[p, :, :] = reduced
            out_ref[pl.ds(k * chunk + p * sub, sub), :] = reduced
            for d in SEND_ORDER:
                t = (k + d) % N_DEV
                r = pltpu.make_async_remote_copy(
                    src_ref=acc_buf.at[p],
                    dst_ref=out_ref.at[pl.ds(k * chunk + p * sub, sub), :],
                    send_sem=ag_send_sems.at[p * npeer + d - 1],
                    recv_sem=ag_recv_sems.at[p * npeer + (N_DEV - 1 - d)],
                    device_id=(t,),
                    device_id_type=pl.DeviceIdType.MESH,
                )
                r.start()
                ag[p][d - 1] = r

        for p in range(P):
            for r in rs[p]:
                r.wait_send()
            for d in WAIT_ORDER:
                ag[p][d - 1].wait_recv()
            for r in ag[p]:
                r.wait_send()

    return pl.pallas_call(
        body,
        out_shape=jax.ShapeDtypeStruct((m, n), x.dtype),
        in_specs=[pl.BlockSpec(memory_space=pltpu.VMEM)],
        out_specs=pl.BlockSpec(memory_space=pltpu.VMEM),
        scratch_shapes=[
            pltpu.VMEM((P, sub, n), x.dtype),
            pltpu.VMEM((P * npeer, sub, n), x.dtype),
            pltpu.SemaphoreType.DMA((P * npeer,)),
            pltpu.SemaphoreType.DMA((P * npeer,)),
            pltpu.SemaphoreType.DMA((P * npeer,)),
            pltpu.SemaphoreType.DMA((P * npeer,)),
        ],
        compiler_params=pltpu.CompilerParams(collective_id=0),
    )(x)


# device time: 11094 ns/iter; 1.0228x vs baseline; 1.0033x over previous
import jax
import jax.numpy as jnp
from jax import lax
from jax.experimental import pallas as pl
from jax.experimental.pallas import tpu as pltpu

N_DEV = 4
P = 2
SEND_ORDER = (2, 1, 3)
WAIT_ORDER = (1, 3, 2)


def kernel(x):
    m, n = x.shape
    chunk = m // N_DEV
    sub = chunk // P
    npeer = N_DEV - 1

    def body(
        x_ref, out_ref,
        rs_recv,
        rs_send_sems, rs_recv_sems, ag_send_sems, ag_recv_sems,
    ):
        k = lax.axis_index("i")

        barrier_sem = pltpu.get_barrier_semaphore()
        for d in range(1, N_DEV):
            pl.semaphore_signal(
                barrier_sem, inc=1,
                device_id=((k + d) % N_DEV,),
                device_id_type=pl.DeviceIdType.MESH,
            )
        pl.semaphore_wait(barrier_sem, npeer)

        rs = [[None] * npeer for _ in range(P)]
        for p in range(P):
            for d in SEND_ORDER:
                t = (k + d) % N_DEV
                q_recv = p * npeer + (N_DEV - 1 - d)
                r = pltpu.make_async_remote_copy(
                    src_ref=x_ref.at[pl.ds(t * chunk + p * sub, sub), :],
                    dst_ref=rs_recv.at[q_recv],
                    send_sem=rs_send_sems.at[p * npeer + d - 1],
                    recv_sem=rs_recv_sems.at[q_recv],
                    device_id=(t,),
                    device_id_type=pl.DeviceIdType.MESH,
                )
                r.start()
                rs[p][d - 1] = r

        ag = [[None] * npeer for _ in range(P)]
        for p in range(P):
            reduced = x_ref[pl.ds(k * chunk + p * sub, sub), :]
            for d in WAIT_ORDER:
                rs[p][d - 1].wait_recv()
                reduced = reduced + rs_recv[p * npeer + (N_DEV - 1 - d), :, :]
            out_ref[pl.ds(k * chunk + p * sub, sub), :] = reduced
            for d in SEND_ORDER:
                t = (k + d) % N_DEV
                r = pltpu.make_async_remote_copy(
                    src_ref=out_ref.at[pl.ds(k * chunk + p * sub, sub), :],
                    dst_ref=out_ref.at[pl.ds(k * chunk + p * sub, sub), :],
                    send_sem=ag_send_sems.at[p * npeer + d - 1],
                    recv_sem=ag_recv_sems.at[p * npeer + (N_DEV - 1 - d)],
                    device_id=(t,),
                    device_id_type=pl.DeviceIdType.MESH,
                )
                r.start()
                ag[p][d - 1] = r

        for p in range(P):
            for r in rs[p]:
                r.wait_send()
            for d in WAIT_ORDER:
                ag[p][d - 1].wait_recv()
            for r in ag[p]:
                r.wait_send()

    return pl.pallas_call(
        body,
        out_shape=jax.ShapeDtypeStruct((m, n), x.dtype),
        in_specs=[pl.BlockSpec(memory_space=pltpu.VMEM)],
        out_specs=pl.BlockSpec(memory_space=pltpu.VMEM),
        scratch_shapes=[
            pltpu.VMEM((P * npeer, sub, n), x.dtype),
            pltpu.SemaphoreType.DMA((P * npeer,)),
            pltpu.SemaphoreType.DMA((P * npeer,)),
            pltpu.SemaphoreType.DMA((P * npeer,)),
            pltpu.SemaphoreType.DMA((P * npeer,)),
        ],
        compiler_params=pltpu.CompilerParams(collective_id=0),
    )(x)
